# baseline (speedup 1.0000x reference)
import functools

import jax
import jax.numpy as jnp
from jax import lax
from jax.experimental import pallas as pl
from jax.experimental.pallas import tpu as pltpu
from jax.experimental.pallas import tpu_sc as plsc

VOCAB = 1000000
D = 64
BATCH = 16384
HIST = 50
N = BATCH * HIST


_info = plsc.get_sparse_core_info()
_NC = _info.num_cores
_NS = _info.num_subcores
_NW = _NC * _NS
_N_PER_W = N // _NW
_CH = 1024
_NCHUNK = _N_PER_W // _CH


def _make_sc_gather():
    mesh = plsc.VectorSubcoreMesh(core_axis_name="c", subcore_axis_name="s")

    @functools.partial(
        pl.kernel,
        mesh=mesh,
        out_type=jax.ShapeDtypeStruct((N, D), jnp.float32),
        scratch_types=[
            pltpu.VMEM((_CH,), jnp.int32),
            pltpu.VMEM((_CH, D), jnp.float32),
            pltpu.SemaphoreType.DMA,
        ],
        compiler_params=pltpu.CompilerParams(use_tc_tiling_on_sc=False),
    )
    def sc_gather(idx_hbm, table_hbm, out_hbm, idx_v, rows_v, sem):
        wid = lax.axis_index("s") * _NC + lax.axis_index("c")
        base = wid * _N_PER_W

        def body(i, carry):
            off = base + i * _CH
            pltpu.sync_copy(idx_hbm.at[pl.ds(off, _CH)], idx_v)
            pltpu.async_copy(table_hbm.at[idx_v], rows_v, sem).wait()
            pltpu.sync_copy(rows_v, out_hbm.at[pl.ds(off, _CH)])
            return carry

        lax.fori_loop(0, _NCHUNK, body, 0)

    return sc_gather


_sc_gather = _make_sc_gather()


_BB = 8
_GR = _BB * HIST // 2


def _mm_body(g_ref, w2_ref, b2_ref, o_ref):
    y = lax.dot_general(g_ref[...], w2_ref[...], (((1,), (0,)), ((), ())),
                        preferred_element_type=jnp.float32) + b2_ref[...]
    ya = y[:, :D]
    yb = y[:, D:]
    for k in range(_BB // 2):
        o_ref[k] = ya[k * HIST:(k + 1) * HIST, :]
        o_ref[_BB // 2 + k] = yb[k * HIST:(k + 1) * HIST, :]


def _tc_matmul(g128, W2, b2):
    grid = (BATCH // _BB,)
    return pl.pallas_call(
        _mm_body,
        grid=grid,
        in_specs=[
            pl.BlockSpec((_GR, 2 * D), lambda i: (i, 0)),
            pl.BlockSpec((2 * D, 2 * D), lambda i: (0, 0)),
            pl.BlockSpec((1, 2 * D), lambda i: (0, 0)),
        ],
        out_specs=pl.BlockSpec((_BB, HIST, D), lambda i: (i, 0, 0)),
        out_shape=jax.ShapeDtypeStruct((BATCH, HIST, D), jnp.float32),
    )(g128, W2, b2)


def kernel(x, emb_table, W, b):
    idx = x.reshape(-1).astype(jnp.int32)
    idx = idx.reshape(-1, 2, _GR).swapaxes(1, 2).reshape(-1)
    g = _sc_gather(idx, emb_table)
    z = jnp.zeros((D, D), jnp.float32)
    wt = W.T
    W2 = jnp.block([[wt, z], [z, wt]])
    b2 = jnp.concatenate([b, b]).reshape(1, 2 * D)
    return _tc_matmul(g.reshape(N // 2, 2 * D), W2, b2)


# speedup vs baseline: 1.9751x; 1.9751x over previous
import functools

import jax
import jax.numpy as jnp
from jax import lax
from jax.experimental import pallas as pl
from jax.experimental.pallas import tpu as pltpu
from jax.experimental.pallas import tpu_sc as plsc

VOCAB = 1000000
D = 64
BATCH = 16384
HIST = 50
N = BATCH * HIST


_PRE_BLK = 10000
_H2 = _PRE_BLK // 2


def _pre_body(t_ref, w_ref, b_ref, o_ref):
    dn = (((1,), (1,)), ((), ()))
    x = t_ref[...]
    ya = lax.dot_general(x[:_H2], w_ref[...], dn,
                         preferred_element_type=jnp.float32) + b_ref[...]
    yb = lax.dot_general(x[_H2:], w_ref[...], dn,
                         preferred_element_type=jnp.float32) + b_ref[...]
    o_ref[...] = jnp.concatenate([ya, yb], axis=1)


def _tc_pretransform(emb_table, W, b2d):
    grid = (VOCAB // _PRE_BLK,)
    return pl.pallas_call(
        _pre_body,
        grid=grid,
        in_specs=[
            pl.BlockSpec((_PRE_BLK, D), lambda i: (i, 0)),
            pl.BlockSpec((D, D), lambda i: (0, 0)),
            pl.BlockSpec((1, D), lambda i: (0, 0)),
        ],
        out_specs=pl.BlockSpec((_H2, 2 * D), lambda i: (i, 0)),
        out_shape=jax.ShapeDtypeStruct((VOCAB // 2, 2 * D), jnp.float32),
    )(emb_table, W, b2d)


_info = plsc.get_sparse_core_info()
_NC = _info.num_cores
_NS = _info.num_subcores
_NW = _NC * _NS
_N_PER_W = N // _NW
_CH = 1024
_NCHUNK = _N_PER_W // _CH


def _make_sc_gather():
    mesh = plsc.VectorSubcoreMesh(core_axis_name="c", subcore_axis_name="s")

    @functools.partial(
        pl.kernel,
        mesh=mesh,
        out_type=jax.ShapeDtypeStruct((N, D), jnp.float32),
        scratch_types=[
            pltpu.VMEM((_CH,), jnp.int32),
            pltpu.VMEM((_CH, D), jnp.float32),
            pltpu.SemaphoreType.DMA,
        ],
        compiler_params=pltpu.CompilerParams(use_tc_tiling_on_sc=False),
    )
    def sc_gather(idx_hbm, table_hbm, out_hbm, idx_v, rows_v, sem):
        wid = lax.axis_index("s") * _NC + lax.axis_index("c")
        base = wid * _N_PER_W

        def body(i, carry):
            off = base + i * _CH
            pltpu.sync_copy(idx_hbm.at[pl.ds(off, _CH)], idx_v)
            pltpu.async_copy(table_hbm.at[idx_v], rows_v, sem).wait()
            pltpu.sync_copy(rows_v, out_hbm.at[pl.ds(off, _CH)])
            return carry

        lax.fori_loop(0, _NCHUNK, body, 0)

    return sc_gather


_sc_gather = _make_sc_gather()


def kernel(x, emb_table, W, b):
    idx = x.reshape(-1).astype(jnp.int32)
    blk_base = (idx // _PRE_BLK) * _PRE_BLK
    loc = idx - blk_base
    idx = blk_base + jnp.where(loc < _H2, 2 * loc, 2 * loc - _PRE_BLK + 1)
    table2 = _tc_pretransform(emb_table, W, b.reshape(1, D))
    g = _sc_gather(idx, table2.reshape(VOCAB, D))
    return g.reshape(BATCH, HIST, D)
